# Initial kernel scaffold; baseline (speedup 1.0000x reference)
import functools

import jax
import jax.numpy as jnp
from jax import lax
from jax.experimental import pallas as pl
from jax.experimental.pallas import tpu as pltpu
from jax.experimental.pallas import tpu_sc as plsc

NC = 2
NS = 16
NW = NC * NS
CHUNK = 128
EMB = 32
HID = 64
OUT = 32


def _make_gather(n_rows: int):
    rows_per_w = n_rows // NW
    nchunk = rows_per_w // CHUNK
    mesh = plsc.VectorSubcoreMesh(
        core_axis_name="c", subcore_axis_name="s", num_cores=NC, num_subcores=NS
    )

    @functools.partial(
        pl.kernel,
        out_type=jax.ShapeDtypeStruct((n_rows, EMB), jnp.float32),
        mesh=mesh,
        scratch_types=[
            pltpu.VMEM((nchunk, CHUNK), jnp.int32),
            pltpu.VMEM((2, CHUNK, EMB), jnp.float32),
            pltpu.SemaphoreType.DMA,
            pltpu.SemaphoreType.DMA,
        ],
    )
    def gather_k(table_hbm, idx_hbm, out_hbm, idx_v, rows_v, sem0, sem1):
        wid = lax.axis_index("s") * NC + lax.axis_index("c")
        base = wid * rows_per_w
        pltpu.sync_copy(idx_hbm.at[wid], idx_v)
        sems = (sem0, sem1)

        pltpu.async_copy(table_hbm.at[idx_v.at[0]], rows_v.at[0], sem0)
        pltpu.async_copy(table_hbm.at[idx_v.at[1]], rows_v.at[1], sem1)

        def body(g, carry):
            for b in range(2):
                j = 2 * g + b
                pltpu.make_async_copy(
                    table_hbm.at[idx_v.at[j]], rows_v.at[b], sems[b]
                ).wait()
                pltpu.sync_copy(
                    rows_v.at[b], out_hbm.at[pl.ds(base + j * CHUNK, CHUNK)]
                )
                pltpu.async_copy(
                    table_hbm.at[idx_v.at[j + 2]], rows_v.at[b], sems[b]
                )
            return carry

        lax.fori_loop(0, nchunk // 2 - 1, body, 0)

        for b in range(2):
            j = nchunk - 2 + b
            pltpu.make_async_copy(
                table_hbm.at[idx_v.at[j]], rows_v.at[b], sems[b]
            ).wait()
            pltpu.sync_copy(
                rows_v.at[b], out_hbm.at[pl.ds(base + j * CHUNK, CHUNK)]
            )

    return gather_k


def _mlp_body(x_ref, w1_ref, b1_ref, w2_ref, b2_ref, o_ref):
    x = x_ref[...]
    pre1 = jnp.dot(x, w1_ref[...], preferred_element_type=jnp.float32) + b1_ref[...]
    h = 1.0 / (1.0 + jnp.exp(-pre1))
    pre2 = jnp.dot(h, w2_ref[...], preferred_element_type=jnp.float32) + b2_ref[...]
    o_ref[...] = 1.0 / (1.0 + jnp.exp(-pre2))


def _mlp(x, W1, b1, W2, b2, blk: int):
    n = x.shape[0]
    return pl.pallas_call(
        _mlp_body,
        grid=(n // blk,),
        in_specs=[
            pl.BlockSpec((blk, EMB), lambda i: (i, 0)),
            pl.BlockSpec((EMB, HID), lambda i: (0, 0)),
            pl.BlockSpec((1, HID), lambda i: (0, 0)),
            pl.BlockSpec((HID, OUT), lambda i: (0, 0)),
            pl.BlockSpec((1, OUT), lambda i: (0, 0)),
        ],
        out_specs=pl.BlockSpec((blk, OUT), lambda i: (i, 0)),
        out_shape=jax.ShapeDtypeStruct((n, OUT), jnp.float32),
    )(x, W1, b1, W2, b2)


def kernel(cards_id, emb_table, W1, b1, W2, b2):
    batch, hist = cards_id.shape
    n_rows = batch * hist
    assert n_rows % (NW * CHUNK) == 0
    idx = cards_id.reshape(-1).astype(jnp.int32).reshape(NW, -1, CHUNK)
    gather_k = _make_gather(n_rows)
    gathered = gather_k(emb_table, idx)
    out = _mlp(gathered, W1, b1.reshape(1, HID), W2, b2.reshape(1, OUT), blk=8192)
    return out.reshape(batch, hist, OUT)


# speedup vs baseline: 9.9153x; 9.9153x over previous
import functools

import jax
import jax.numpy as jnp
from jax import lax
from jax.experimental import pallas as pl
from jax.experimental.pallas import tpu as pltpu
from jax.experimental.pallas import tpu_sc as plsc

NC = 2
NS = 16
NW = NC * NS
CHUNK = 128
EMB = 32
HID = 64
OUT = 32


def _make_gather(n_rows: int):
    rows_per_w = n_rows // NW
    nchunk = rows_per_w // CHUNK
    mesh = plsc.VectorSubcoreMesh(
        core_axis_name="c", subcore_axis_name="s", num_cores=NC, num_subcores=NS
    )

    @functools.partial(
        pl.kernel,
        out_type=jax.ShapeDtypeStruct((n_rows, EMB), jnp.float32),
        mesh=mesh,
        scratch_types=[
            pltpu.VMEM((nchunk, CHUNK), jnp.int32),
            pltpu.VMEM((2, CHUNK, EMB), jnp.float32),
            pltpu.SemaphoreType.DMA,
            pltpu.SemaphoreType.DMA,
        ],
        compiler_params=pltpu.CompilerParams(use_tc_tiling_on_sc=False),
    )
    def gather_k(table_hbm, idx_hbm, out_hbm, idx_v, rows_v, sem0, sem1):
        wid = lax.axis_index("s") * NC + lax.axis_index("c")
        base = wid * rows_per_w
        pltpu.sync_copy(idx_hbm.at[wid], idx_v)
        sems = (sem0, sem1)

        pltpu.async_copy(table_hbm.at[idx_v.at[0]], rows_v.at[0], sem0)
        pltpu.async_copy(table_hbm.at[idx_v.at[1]], rows_v.at[1], sem1)

        def body(g, carry):
            for b in range(2):
                j = 2 * g + b
                pltpu.make_async_copy(
                    table_hbm.at[idx_v.at[j]], rows_v.at[b], sems[b]
                ).wait()
                pltpu.sync_copy(
                    rows_v.at[b], out_hbm.at[pl.ds(base + j * CHUNK, CHUNK)]
                )
                pltpu.async_copy(
                    table_hbm.at[idx_v.at[j + 2]], rows_v.at[b], sems[b]
                )
            return carry

        lax.fori_loop(0, nchunk // 2 - 1, body, 0)

        for b in range(2):
            j = nchunk - 2 + b
            pltpu.make_async_copy(
                table_hbm.at[idx_v.at[j]], rows_v.at[b], sems[b]
            ).wait()
            pltpu.sync_copy(
                rows_v.at[b], out_hbm.at[pl.ds(base + j * CHUNK, CHUNK)]
            )

    return gather_k


def _mlp_body(x_ref, w1_ref, b1_ref, w2_ref, b2_ref, o_ref):
    x = x_ref[...]
    pre1 = jnp.dot(x, w1_ref[...], preferred_element_type=jnp.float32) + b1_ref[...]
    h = 1.0 / (1.0 + jnp.exp(-pre1))
    pre2 = jnp.dot(h, w2_ref[...], preferred_element_type=jnp.float32) + b2_ref[...]
    o_ref[...] = 1.0 / (1.0 + jnp.exp(-pre2))


def _mlp(x, W1, b1, W2, b2, blk: int):
    n = x.shape[0]
    return pl.pallas_call(
        _mlp_body,
        grid=(n // blk,),
        in_specs=[
            pl.BlockSpec((blk, EMB), lambda i: (i, 0)),
            pl.BlockSpec((EMB, HID), lambda i: (0, 0)),
            pl.BlockSpec((1, HID), lambda i: (0, 0)),
            pl.BlockSpec((HID, OUT), lambda i: (0, 0)),
            pl.BlockSpec((1, OUT), lambda i: (0, 0)),
        ],
        out_specs=pl.BlockSpec((blk, OUT), lambda i: (i, 0)),
        out_shape=jax.ShapeDtypeStruct((n, OUT), jnp.float32),
    )(x, W1, b1, W2, b2)


def kernel(cards_id, emb_table, W1, b1, W2, b2):
    batch, hist = cards_id.shape
    n_rows = batch * hist
    assert n_rows % (NW * CHUNK) == 0
    idx = cards_id.reshape(-1).astype(jnp.int32).reshape(NW, -1, CHUNK)
    gather_k = _make_gather(n_rows)
    gathered = gather_k(emb_table, idx)
    out = _mlp(gathered, W1, b1.reshape(1, HID), W2, b2.reshape(1, OUT), blk=8192)
    return out.reshape(batch, hist, OUT)


# speedup vs baseline: 23.0949x; 2.3292x over previous
import functools

import jax
import jax.numpy as jnp
from jax import lax
from jax.experimental import pallas as pl
from jax.experimental.pallas import tpu as pltpu
from jax.experimental.pallas import tpu_sc as plsc

NC = 2
NS = 16
NW = NC * NS
EMB = 32
HID = 64
OUT = 32
UNIT = 128
LANE = 128
GRP = LANE // EMB
MROW = 1024
BBT = GRP * MROW


def _make_gather(batch: int, hist: int):
    n_rows = batch * hist
    cols_per_w = batch // NW
    tunits = cols_per_w // UNIT
    nunit = hist * tunits
    nj = batch // BBT
    mesh = plsc.VectorSubcoreMesh(
        core_axis_name="c", subcore_axis_name="s", num_cores=NC, num_subcores=NS
    )

    @functools.partial(
        pl.kernel,
        out_type=jax.ShapeDtypeStruct((n_rows // GRP, LANE), jnp.float32),
        mesh=mesh,
        scratch_types=[
            pltpu.VMEM((hist, cols_per_w), jnp.int32),
            pltpu.VMEM((2, UNIT, EMB), jnp.float32),
            pltpu.SemaphoreType.DMA,
            pltpu.SemaphoreType.DMA,
        ],
        compiler_params=pltpu.CompilerParams(use_tc_tiling_on_sc=False),
    )
    def gather_k(table_hbm, idx_hbm, out_hbm, idx_v, rows_v, sem0, sem1):
        wid = lax.axis_index("s") * NC + lax.axis_index("c")
        col0 = wid * cols_per_w
        jblk = wid // 8
        jgrp = (wid % 8) // 2
        half = (wid % 2) * 512
        lane0 = jgrp * EMB
        pltpu.sync_copy(idx_hbm.at[:, pl.ds(col0, cols_per_w)], idx_v)
        sems = (sem0, sem1)

        def unit_src(u):
            l = u // tunits
            t = u % tunits
            return table_hbm.at[idx_v.at[l, pl.ds(t * UNIT, UNIT)]]

        def unit_dst(u):
            l = u // tunits
            t = u % tunits
            q0 = (l * nj + jblk) * MROW + half + t * UNIT
            return out_hbm.at[pl.ds(q0, UNIT), pl.ds(lane0, EMB)]

        for b in range(2):
            pltpu.async_copy(unit_src(b), rows_v.at[b], sems[b])

        def body(t, carry):
            for b in range(2):
                u = 2 * t + b
                pltpu.make_async_copy(unit_src(u), rows_v.at[b], sems[b]).wait()
                pltpu.sync_copy(rows_v.at[b], unit_dst(u))
                pltpu.async_copy(unit_src(u + 2), rows_v.at[b], sems[b])
            return carry

        lax.fori_loop(0, nunit // 2 - 1, body, 0)

        for b in range(2):
            u = nunit - 2 + b
            pltpu.make_async_copy(unit_src(u), rows_v.at[b], sems[b]).wait()
            pltpu.sync_copy(rows_v.at[b], unit_dst(u))

    return gather_k


def _mlp_body(x_ref, w1_ref, b1_ref, w2_ref, b2_ref, o_ref):
    xp = x_ref[...]
    w1 = w1_ref[...]
    b1 = b1_ref[...]
    w2 = w2_ref[...]
    b2 = b2_ref[...]
    for j in range(GRP):
        x = xp[:, j * EMB:(j + 1) * EMB]
        pre1 = jnp.dot(x, w1, preferred_element_type=jnp.float32) + b1
        h = 1.0 / (1.0 + jnp.exp(-pre1))
        pre2 = jnp.dot(h, w2, preferred_element_type=jnp.float32) + b2
        y = 1.0 / (1.0 + jnp.exp(-pre2))
        o_ref[0, :, pl.ds(j * MROW, MROW)] = y.T


def _mlp(x_packed, W1, b1, W2, b2, batch, hist):
    nj = batch // BBT
    return pl.pallas_call(
        _mlp_body,
        grid=(hist, nj),
        in_specs=[
            pl.BlockSpec((MROW, LANE), lambda l, j: (l * nj + j, 0)),
            pl.BlockSpec((EMB, HID), lambda l, j: (0, 0)),
            pl.BlockSpec((1, HID), lambda l, j: (0, 0)),
            pl.BlockSpec((HID, OUT), lambda l, j: (0, 0)),
            pl.BlockSpec((1, OUT), lambda l, j: (0, 0)),
        ],
        out_specs=pl.BlockSpec((1, OUT, BBT), lambda l, j: (l, 0, j)),
        out_shape=jax.ShapeDtypeStruct((hist, OUT, batch), jnp.float32),
    )(x_packed, W1, b1, W2, b2)


def kernel(cards_id, emb_table, W1, b1, W2, b2):
    batch, hist = cards_id.shape
    assert batch % (NW * UNIT) == 0 and batch % BBT == 0
    idx_t = cards_id.T.astype(jnp.int32)
    gather_k = _make_gather(batch, hist)
    packed = gather_k(emb_table, idx_t)
    out_t = _mlp(
        packed, W1, b1.reshape(1, HID), W2, b2.reshape(1, OUT), batch, hist,
    )
    return out_t.transpose(2, 0, 1)
